# baseline (speedup 1.0000x reference)
import functools

import jax
import jax.numpy as jnp
from jax import lax
from jax.experimental import pallas as pl
from jax.experimental.pallas import tpu as pltpu
from jax.experimental.pallas import tpu_sc as plsc

_L = 16
_REP = 81
_SCH = 8


def _bucket(cv):
    b = jnp.where(cv >= 1, jnp.int32(1), jnp.int32(0))
    b = jnp.where(cv >= 11, jnp.int32(2), b)
    b = jnp.where(cv >= 37, jnp.int32(3), b)
    b = jnp.where(cv >= 63, jnp.int32(4), b)
    return b


def _make_sc_lookup(bsz, seq, emb):
    nc, ns = 2, 16
    nw = nc * ns
    nbb = bsz // 128
    bb_per_w = nbb // nw
    n_sch = seq // _SCH
    n_chunks = bb_per_w * n_sch
    eb = emb // 8

    mesh = plsc.VectorSubcoreMesh(
        core_axis_name="c",
        subcore_axis_name="s",
        num_cores=nc,
        num_subcores=ns,
    )

    @functools.partial(
        pl.kernel,
        mesh=mesh,
        out_type=jax.ShapeDtypeStruct((seq, eb, nbb, 8, 128), jnp.float32),
        compiler_params=pltpu.CompilerParams(needs_layout_passes=False),
        scratch_types=[
            pltpu.VMEM((2, _SCH, 128), jnp.int32),
            pltpu.VMEM((2, _SCH, eb, 8, 128), jnp.float32),
            pltpu.VMEM((5, emb), jnp.float32),
            pltpu.VMEM((_L * _REP,), jnp.float32),
            pltpu.SemaphoreType.DMA,
            pltpu.SemaphoreType.DMA,
            pltpu.SemaphoreType.DMA,
            pltpu.SemaphoreType.DMA,
        ],
    )
    def sc_lookup(chars_hbm, w_hbm, out_hbm, ids_v, rows_v, w_v, wt_v,
                  si0, si1, so0, so1):
        wid = lax.axis_index("s") * nc + lax.axis_index("c")
        bb0 = wid * bb_per_w
        sin = (si0, si1)
        sout = (so0, so1)
        pltpu.sync_copy(w_hbm, w_v)
        for c in range(_L):
            for b in range(5):
                wt_v[pl.ds(c * _REP + b * emb, emb)] = w_v[b, :]
        lane81 = lax.iota(jnp.int32, _L) * _REP

        def in_src(k):
            s0 = (k >> 2) * _SCH
            bb = bb0 + (k & 3)
            return chars_hbm.at[pl.ds(s0, _SCH), pl.ds(bb * 128, 128)]

        def out_dst(k):
            s0 = (k >> 2) * _SCH
            bb = bb0 + (k & 3)
            return out_hbm.at[pl.ds(s0, _SCH), :, bb]

        def compute(b):
            def s_body(si, _):
                for bg in range(8):
                    cv = ids_v[b, si, pl.ds(bg * _L, _L)]
                    fb = _bucket(cv) * emb + lane81
                    for e in range(emb):
                        row = plsc.load_gather(wt_v, [fb + e])
                        rows_v[b, si, e // 8, e % 8, pl.ds(bg * _L, _L)] = row
                return 0

            lax.fori_loop(0, _SCH, s_body, 0, unroll=2)

        cp_in = [pltpu.async_copy(in_src(b), ids_v.at[b], sin[b])
                 for b in (0, 1)]

        for b in (0, 1):
            cp_in[b].wait()
            compute(b)
            pltpu.async_copy(rows_v.at[b], out_dst(b), sout[b])
            pltpu.async_copy(in_src(b + 2), ids_v.at[b], sin[b])

        def pair_body(m, _):
            for b in (0, 1):
                k = 2 * m + b
                pltpu.make_async_copy(in_src(k), ids_v.at[b], sin[b]).wait()
                pltpu.make_async_copy(rows_v.at[b], out_dst(k - 2),
                                      sout[b]).wait()
                compute(b)
                pltpu.async_copy(rows_v.at[b], out_dst(k), sout[b])

                @pl.when(k + 2 < n_chunks)
                def _():
                    pltpu.async_copy(in_src(k + 2), ids_v.at[b], sin[b])

            return 0

        lax.fori_loop(1, n_chunks // 2, pair_body, 0, unroll=False)

        for b in (0, 1):
            k = n_chunks - 2 + b
            pltpu.make_async_copy(rows_v.at[b], out_dst(k), sout[b]).wait()

    return sc_lookup


def kernel(chars, embedding_weight):
    bsz, seq = chars.shape
    n_cls, emb = embedding_weight.shape
    out5 = _make_sc_lookup(bsz, seq, emb)(chars.T, embedding_weight)
    return out5.transpose(2, 4, 0, 1, 3).reshape(bsz, seq, emb)


# speedup vs baseline: 2.3400x; 2.3400x over previous
import functools

import jax
import jax.numpy as jnp
from jax import lax
from jax.experimental import pallas as pl
from jax.experimental.pallas import tpu as pltpu
from jax.experimental.pallas import tpu_sc as plsc

_L = 16
_REP = 81
_SCH = 8


def _bucket(cv):
    b = jnp.where(cv >= 1, jnp.int32(1), jnp.int32(0))
    b = jnp.where(cv >= 11, jnp.int32(2), b)
    b = jnp.where(cv >= 37, jnp.int32(3), b)
    b = jnp.where(cv >= 63, jnp.int32(4), b)
    return b


def _make_sc_lookup(bsz, seq, emb):
    nc, ns = 2, 16
    nw = nc * ns
    nbb = bsz // 128
    bb_per_w = nbb // nw
    n_sch = seq // _SCH
    n_chunks = bb_per_w * n_sch
    eb = emb // 8

    mesh = plsc.VectorSubcoreMesh(
        core_axis_name="c",
        subcore_axis_name="s",
        num_cores=nc,
        num_subcores=ns,
    )

    @functools.partial(
        pl.kernel,
        mesh=mesh,
        out_type=jax.ShapeDtypeStruct((seq, eb, nbb, 8, 128), jnp.float32),
        compiler_params=pltpu.CompilerParams(needs_layout_passes=False),
        scratch_types=[
            pltpu.VMEM((2, _SCH, 128), jnp.int32),
            pltpu.VMEM((2, _SCH, eb, 8, 128), jnp.float32),
            pltpu.VMEM((5, emb), jnp.float32),
            pltpu.VMEM((_L * _REP,), jnp.float32),
            pltpu.SemaphoreType.DMA,
            pltpu.SemaphoreType.DMA,
            pltpu.SemaphoreType.DMA,
            pltpu.SemaphoreType.DMA,
        ],
    )
    def sc_lookup(chars_hbm, w_hbm, out_hbm, ids_v, rows_v, w_v, wt_v,
                  si0, si1, so0, so1):
        wid = lax.axis_index("s") * nc + lax.axis_index("c")
        bb0 = wid * bb_per_w
        sin = (si0, si1)
        sout = (so0, so1)
        pltpu.sync_copy(w_hbm, w_v)
        for c in range(_L):
            for b in range(5):
                wt_v[pl.ds(c * _REP + b * emb, emb)] = w_v[b, :]
        lane81 = lax.iota(jnp.int32, _L) * _REP

        def in_src(k):
            s0 = (k >> 2) * _SCH
            bb = bb0 + (k & 3)
            return chars_hbm.at[pl.ds(s0, _SCH), pl.ds(bb * 128, 128)]

        def out_dst(k):
            s0 = (k >> 2) * _SCH
            bb = bb0 + (k & 3)
            return out_hbm.at[pl.ds(s0, _SCH), :, bb]

        def compute(b):
            def s_body(si, _):
                for bg in range(8):
                    cv = ids_v[b, si, pl.ds(bg * _L, _L)]
                    fb = _bucket(cv) * emb + lane81
                    rows = [plsc.load_gather(wt_v, [fb + e])
                            for e in range(emb)]
                    for e in range(emb):
                        rows_v[b, si, e // 8, e % 8, pl.ds(bg * _L, _L)] = (
                            rows[e])
                return 0

            lax.fori_loop(0, _SCH, s_body, 0, unroll=False)

        cp_in = [pltpu.async_copy(in_src(b), ids_v.at[b], sin[b])
                 for b in (0, 1)]

        for b in (0, 1):
            cp_in[b].wait()
            compute(b)
            pltpu.async_copy(rows_v.at[b], out_dst(b), sout[b])
            pltpu.async_copy(in_src(b + 2), ids_v.at[b], sin[b])

        def pair_body(m, _):
            for b in (0, 1):
                k = 2 * m + b
                pltpu.make_async_copy(in_src(k), ids_v.at[b], sin[b]).wait()
                pltpu.make_async_copy(rows_v.at[b], out_dst(k - 2),
                                      sout[b]).wait()
                compute(b)
                pltpu.async_copy(rows_v.at[b], out_dst(k), sout[b])

                @pl.when(k + 2 < n_chunks)
                def _():
                    pltpu.async_copy(in_src(k + 2), ids_v.at[b], sin[b])

            return 0

        lax.fori_loop(1, n_chunks // 2, pair_body, 0, unroll=False)

        for b in (0, 1):
            k = n_chunks - 2 + b
            pltpu.make_async_copy(rows_v.at[b], out_dst(k), sout[b]).wait()

    return sc_lookup


def kernel(chars, embedding_weight):
    bsz, seq = chars.shape
    n_cls, emb = embedding_weight.shape
    out5 = _make_sc_lookup(bsz, seq, emb)(chars.T, embedding_weight)
    return out5.transpose(2, 4, 0, 1, 3).reshape(bsz, seq, emb)


# speedup vs baseline: 3.1707x; 1.3550x over previous
import functools

import jax
import jax.numpy as jnp
from jax import lax
from jax.experimental import pallas as pl
from jax.experimental.pallas import tpu as pltpu
from jax.experimental.pallas import tpu_sc as plsc

_L = 16
_REP = 81
_SCH = 8


def _bucket(cv):
    b = jnp.where(cv >= 1, jnp.int32(1), jnp.int32(0))
    b = jnp.where(cv >= 11, jnp.int32(2), b)
    b = jnp.where(cv >= 37, jnp.int32(3), b)
    b = jnp.where(cv >= 63, jnp.int32(4), b)
    return b


def _make_sc_lookup(bsz, seq, emb):
    nc, ns = 2, 16
    nw = nc * ns
    nbb = bsz // 128
    bb_per_w = nbb // nw
    n_sch = seq // _SCH
    n_chunks = bb_per_w * n_sch
    eb = emb // 8

    mesh = plsc.VectorSubcoreMesh(
        core_axis_name="c",
        subcore_axis_name="s",
        num_cores=nc,
        num_subcores=ns,
    )

    @functools.partial(
        pl.kernel,
        mesh=mesh,
        out_type=jax.ShapeDtypeStruct((seq, eb, nbb, 8, 128), jnp.float32),
        compiler_params=pltpu.CompilerParams(needs_layout_passes=False),
        scratch_types=[
            pltpu.VMEM((2, _SCH, 128), jnp.int32),
            pltpu.VMEM((2, _SCH, eb, 8, 128), jnp.float32),
            pltpu.VMEM((5, emb), jnp.float32),
            pltpu.VMEM((_L * _REP,), jnp.float32),
            pltpu.SemaphoreType.DMA,
            pltpu.SemaphoreType.DMA,
            pltpu.SemaphoreType.DMA,
            pltpu.SemaphoreType.DMA,
        ],
    )
    def sc_lookup(chars_hbm, w_hbm, out_hbm, ids_v, rows_v, w_v, wt_v,
                  si0, si1, so0, so1):
        wid = lax.axis_index("s") * nc + lax.axis_index("c")
        bb0 = wid * bb_per_w
        sin = (si0, si1)
        sout = (so0, so1)
        pltpu.sync_copy(w_hbm, w_v)
        for c in range(_L):
            for b in range(5):
                wt_v[pl.ds(c * _REP + b * emb, emb)] = w_v[b, :]
        lane81 = lax.iota(jnp.int32, _L) * _REP

        def in_src(k):
            s0 = (k >> 2) * _SCH
            bb = bb0 + (k & 3)
            return chars_hbm.at[pl.ds(s0, _SCH), pl.ds(bb * 128, 128)]

        def out_dst(k):
            s0 = (k >> 2) * _SCH
            bb = bb0 + (k & 3)
            return out_hbm.at[pl.ds(s0, _SCH), :, bb]

        def compute(b):
            def s_body(si, _):
                for bg2 in range(4):
                    fbs = []
                    for bg in (2 * bg2, 2 * bg2 + 1):
                        cv = ids_v[b, si, pl.ds(bg * _L, _L)]
                        fbs.append(_bucket(cv) * emb + lane81)
                    rows = [[plsc.load_gather(wt_v, [fb + e])
                             for e in range(emb)] for fb in fbs]
                    for g, bg in enumerate((2 * bg2, 2 * bg2 + 1)):
                        for e in range(emb):
                            rows_v[b, si, e // 8, e % 8,
                                   pl.ds(bg * _L, _L)] = rows[g][e]
                return 0

            lax.fori_loop(0, _SCH, s_body, 0, unroll=False)

        cp_in = [pltpu.async_copy(in_src(b), ids_v.at[b], sin[b])
                 for b in (0, 1)]

        for b in (0, 1):
            cp_in[b].wait()
            compute(b)
            pltpu.async_copy(rows_v.at[b], out_dst(b), sout[b])
            pltpu.async_copy(in_src(b + 2), ids_v.at[b], sin[b])

        def pair_body(m, _):
            for b in (0, 1):
                k = 2 * m + b
                pltpu.make_async_copy(in_src(k), ids_v.at[b], sin[b]).wait()
                pltpu.make_async_copy(rows_v.at[b], out_dst(k - 2),
                                      sout[b]).wait()
                compute(b)
                pltpu.async_copy(rows_v.at[b], out_dst(k), sout[b])

                @pl.when(k + 2 < n_chunks)
                def _():
                    pltpu.async_copy(in_src(k + 2), ids_v.at[b], sin[b])

            return 0

        lax.fori_loop(1, n_chunks // 2, pair_body, 0, unroll=False)

        for b in (0, 1):
            k = n_chunks - 2 + b
            pltpu.make_async_copy(rows_v.at[b], out_dst(k), sout[b]).wait()

    return sc_lookup


def kernel(chars, embedding_weight):
    bsz, seq = chars.shape
    n_cls, emb = embedding_weight.shape
    out5 = _make_sc_lookup(bsz, seq, emb)(chars.T, embedding_weight)
    return out5.transpose(2, 4, 0, 1, 3).reshape(bsz, seq, emb)


# speedup vs baseline: 3.1734x; 1.0009x over previous
import functools

import jax
import jax.numpy as jnp
from jax import lax
from jax.experimental import pallas as pl
from jax.experimental.pallas import tpu as pltpu
from jax.experimental.pallas import tpu_sc as plsc

_L = 16
_REP = 81
_SCH = 8


def _bucket(cv):
    b = jnp.where(cv >= 1, jnp.int32(1), jnp.int32(0))
    b = jnp.where(cv >= 11, jnp.int32(2), b)
    b = jnp.where(cv >= 37, jnp.int32(3), b)
    b = jnp.where(cv >= 63, jnp.int32(4), b)
    return b


def _make_sc_lookup(bsz, seq, emb):
    nc, ns = 2, 16
    nw = nc * ns
    nbb = bsz // 128
    bb_per_w = nbb // nw
    n_sch = seq // _SCH
    n_chunks = bb_per_w * n_sch
    eb = emb // 8

    mesh = plsc.VectorSubcoreMesh(
        core_axis_name="c",
        subcore_axis_name="s",
        num_cores=nc,
        num_subcores=ns,
    )

    @functools.partial(
        pl.kernel,
        mesh=mesh,
        out_type=jax.ShapeDtypeStruct((seq, eb, nbb, 8, 128), jnp.float32),
        compiler_params=pltpu.CompilerParams(needs_layout_passes=False),
        scratch_types=[
            pltpu.VMEM((2, _SCH, 128), jnp.int32),
            pltpu.VMEM((2, _SCH, eb, 8, 128), jnp.float32),
            pltpu.VMEM((5, emb), jnp.float32),
            pltpu.VMEM((_L * _REP,), jnp.float32),
            pltpu.SemaphoreType.DMA,
            pltpu.SemaphoreType.DMA,
            pltpu.SemaphoreType.DMA,
            pltpu.SemaphoreType.DMA,
        ],
    )
    def sc_lookup(chars_hbm, w_hbm, out_hbm, ids_v, rows_v, w_v, wt_v,
                  si0, si1, so0, so1):
        wid = lax.axis_index("s") * nc + lax.axis_index("c")
        bb0 = wid * bb_per_w
        sin = (si0, si1)
        sout = (so0, so1)
        pltpu.sync_copy(w_hbm, w_v)
        for c in range(_L):
            for b in range(5):
                wt_v[pl.ds(c * _REP + b * emb, emb)] = w_v[b, :]
        lane81 = lax.iota(jnp.int32, _L) * _REP

        def in_src(k):
            s0 = (k >> 2) * _SCH
            bb = bb0 + (k & 3)
            return chars_hbm.at[pl.ds(s0, _SCH), pl.ds(bb * 128, 128)]

        def out_dst(k):
            s0 = (k >> 2) * _SCH
            bb = bb0 + (k & 3)
            return out_hbm.at[pl.ds(s0, _SCH), :, bb]

        def compute(b):
            def s_body(si, _):
                prev, prev_bg = None, 0
                for bg in range(8):
                    cv = ids_v[b, si, pl.ds(bg * _L, _L)]
                    fb = _bucket(cv) * emb + lane81
                    rows = []
                    for e in range(emb):
                        rows.append(plsc.load_gather(wt_v, [fb + e]))
                        if prev is not None:
                            rows_v[b, si, e // 8, e % 8,
                                   pl.ds(prev_bg * _L, _L)] = prev[e]
                    prev, prev_bg = rows, bg
                for e in range(emb):
                    rows_v[b, si, e // 8, e % 8,
                           pl.ds(prev_bg * _L, _L)] = prev[e]
                return 0

            lax.fori_loop(0, _SCH, s_body, 0, unroll=False)

        cp_in = [pltpu.async_copy(in_src(b), ids_v.at[b], sin[b])
                 for b in (0, 1)]

        for b in (0, 1):
            cp_in[b].wait()
            compute(b)
            pltpu.async_copy(rows_v.at[b], out_dst(b), sout[b])
            pltpu.async_copy(in_src(b + 2), ids_v.at[b], sin[b])

        def pair_body(m, _):
            for b in (0, 1):
                k = 2 * m + b
                pltpu.make_async_copy(in_src(k), ids_v.at[b], sin[b]).wait()
                pltpu.make_async_copy(rows_v.at[b], out_dst(k - 2),
                                      sout[b]).wait()
                compute(b)
                pltpu.async_copy(rows_v.at[b], out_dst(k), sout[b])

                @pl.when(k + 2 < n_chunks)
                def _():
                    pltpu.async_copy(in_src(k + 2), ids_v.at[b], sin[b])

            return 0

        lax.fori_loop(1, n_chunks // 2, pair_body, 0, unroll=False)

        for b in (0, 1):
            k = n_chunks - 2 + b
            pltpu.make_async_copy(rows_v.at[b], out_dst(k), sout[b]).wait()

    return sc_lookup


def kernel(chars, embedding_weight):
    bsz, seq = chars.shape
    n_cls, emb = embedding_weight.shape
    out5 = _make_sc_lookup(bsz, seq, emb)(chars.T, embedding_weight)
    return out5.transpose(2, 4, 0, 1, 3).reshape(bsz, seq, emb)


# speedup vs baseline: 3.1793x; 1.0019x over previous
import functools

import jax
import jax.numpy as jnp
from jax import lax
from jax.experimental import pallas as pl
from jax.experimental.pallas import tpu as pltpu
from jax.experimental.pallas import tpu_sc as plsc

_L = 16
_REP = 81
_SCH = 8


def _bucket(cv):
    b = jnp.where(cv >= 1, jnp.int32(1), jnp.int32(0))
    b = jnp.where(cv >= 11, jnp.int32(2), b)
    b = jnp.where(cv >= 37, jnp.int32(3), b)
    b = jnp.where(cv >= 63, jnp.int32(4), b)
    return b


def _make_sc_lookup(bsz, seq, emb):
    nc, ns = 2, 16
    nw = nc * ns
    nbb = bsz // 128
    bb_per_w = nbb // nw
    n_sch = seq // _SCH
    n_chunks = bb_per_w * n_sch
    eb = emb // 8
    assert bb_per_w == 4 and bsz == nbb * 128
    assert seq % _SCH == 0 and emb == _L and n_chunks % 2 == 0

    mesh = plsc.VectorSubcoreMesh(
        core_axis_name="c",
        subcore_axis_name="s",
        num_cores=nc,
        num_subcores=ns,
    )

    @functools.partial(
        pl.kernel,
        mesh=mesh,
        out_type=jax.ShapeDtypeStruct((seq, eb, nbb, 8, 128), jnp.float32),
        compiler_params=pltpu.CompilerParams(needs_layout_passes=False),
        scratch_types=[
            pltpu.VMEM((2, _SCH, 128), jnp.int32),
            pltpu.VMEM((2, _SCH, eb, 8, 128), jnp.float32),
            pltpu.VMEM((5, emb), jnp.float32),
            pltpu.VMEM((_L * _REP,), jnp.float32),
            pltpu.SemaphoreType.DMA,
            pltpu.SemaphoreType.DMA,
            pltpu.SemaphoreType.DMA,
            pltpu.SemaphoreType.DMA,
        ],
    )
    def sc_lookup(chars_hbm, w_hbm, out_hbm, ids_v, rows_v, w_v, wt_v,
                  si0, si1, so0, so1):
        wid = lax.axis_index("s") * nc + lax.axis_index("c")
        bb0 = wid * bb_per_w
        sin = (si0, si1)
        sout = (so0, so1)
        pltpu.sync_copy(w_hbm, w_v)
        for c in range(_L):
            for b in range(5):
                wt_v[pl.ds(c * _REP + b * emb, emb)] = w_v[b, :]
        lane81 = lax.iota(jnp.int32, _L) * _REP

        def in_src(k):
            s0 = (k >> 2) * _SCH
            bb = bb0 + (k & 3)
            return chars_hbm.at[pl.ds(s0, _SCH), pl.ds(bb * 128, 128)]

        def out_dst(k):
            s0 = (k >> 2) * _SCH
            bb = bb0 + (k & 3)
            return out_hbm.at[pl.ds(s0, _SCH), :, bb]

        def compute(b):
            def s_body(si, _):
                prev, prev_bg = None, 0
                for bg in range(8):
                    cv = ids_v[b, si, pl.ds(bg * _L, _L)]
                    fb = _bucket(cv) * emb + lane81
                    rows = []
                    for e in range(emb):
                        rows.append(plsc.load_gather(wt_v, [fb + e]))
                        if prev is not None:
                            rows_v[b, si, e // 8, e % 8,
                                   pl.ds(prev_bg * _L, _L)] = prev[e]
                    prev, prev_bg = rows, bg
                for e in range(emb):
                    rows_v[b, si, e // 8, e % 8,
                           pl.ds(prev_bg * _L, _L)] = prev[e]
                return 0

            lax.fori_loop(0, _SCH, s_body, 0, unroll=False)

        cp_in = [pltpu.async_copy(in_src(b), ids_v.at[b], sin[b])
                 for b in (0, 1)]

        for b in (0, 1):
            cp_in[b].wait()
            compute(b)
            pltpu.async_copy(rows_v.at[b], out_dst(b), sout[b])
            pltpu.async_copy(in_src(b + 2), ids_v.at[b], sin[b])

        def pair_body(m, _):
            for b in (0, 1):
                k = 2 * m + b
                pltpu.make_async_copy(in_src(k), ids_v.at[b], sin[b]).wait()
                pltpu.make_async_copy(rows_v.at[b], out_dst(k - 2),
                                      sout[b]).wait()
                compute(b)
                pltpu.async_copy(rows_v.at[b], out_dst(k), sout[b])

                @pl.when(k + 2 < n_chunks)
                def _():
                    pltpu.async_copy(in_src(k + 2), ids_v.at[b], sin[b])

            return 0

        lax.fori_loop(1, n_chunks // 2, pair_body, 0, unroll=False)

        for b in (0, 1):
            k = n_chunks - 2 + b
            pltpu.make_async_copy(rows_v.at[b], out_dst(k), sout[b]).wait()

    return sc_lookup


def kernel(chars, embedding_weight):
    bsz, seq = chars.shape
    n_cls, emb = embedding_weight.shape
    out5 = _make_sc_lookup(bsz, seq, emb)(chars.T, embedding_weight)
    return out5.transpose(2, 4, 0, 1, 3).reshape(bsz, seq, emb)
